# baseline (speedup 1.0000x reference)
import functools

import jax
import jax.numpy as jnp
from jax import lax
from jax.experimental import pallas as pl
from jax.experimental.pallas import tpu as pltpu
from jax.experimental.pallas import tpu_sc as plsc

_BATCH = 16384
_FIELDS = 26
_DIM = 64
_NUM_WORKERS = 32
_ROWS_PER_WORKER = _BATCH // _NUM_WORKERS
_GROUP = 16
_NUM_GROUPS = _ROWS_PER_WORKER // _GROUP
_NBUF = 2
_NOUTER = _NUM_GROUPS // _NBUF

_mesh = plsc.VectorSubcoreMesh(core_axis_name="c", subcore_axis_name="s")


@functools.partial(
    pl.kernel,
    out_type=jax.ShapeDtypeStruct((_BATCH, _FIELDS, _DIM), jnp.float32),
    mesh=_mesh,
    scratch_types=[
        pltpu.VMEM((_ROWS_PER_WORKER, _FIELDS), jnp.int32),
        pltpu.VMEM((_NBUF, _GROUP, _FIELDS, _DIM), jnp.float32),
        pltpu.SemaphoreType.DMA,
        pltpu.SemaphoreType.DMA,
        pltpu.SemaphoreType.DMA,
        pltpu.SemaphoreType.DMA,
    ],
    compiler_params=pltpu.CompilerParams(use_tc_tiling_on_sc=False),
)
def _gather_kernel(keys_hbm, table_hbm, out_hbm, idx_v, rows_v, gs0, gs1, os0, os1):
    gsem = (gs0, gs1)
    osem = (os0, os1)
    wid = lax.axis_index("s") * 2 + lax.axis_index("c")
    base = wid * _ROWS_PER_WORKER
    pltpu.sync_copy(keys_hbm.at[pl.ds(base, _ROWS_PER_WORKER)], idx_v)

    def gather(g, b):
        def start():
            for i in range(_GROUP):
                pltpu.make_async_copy(
                    table_hbm.at[idx_v.at[g * _GROUP + i]],
                    rows_v.at[b].at[i], gsem[b]).start()

        def wait():
            for i in range(_GROUP):
                pltpu.make_async_copy(
                    table_hbm.at[idx_v.at[g * _GROUP + i]],
                    rows_v.at[b].at[i], gsem[b]).wait()

        return start, wait

    def store(g, b):
        return pltpu.make_async_copy(
            rows_v.at[b], out_hbm.at[pl.ds(base + g * _GROUP, _GROUP)], osem[b])

    for b in range(_NBUF):
        gather(b, b)[0]()

    def body(i, carry):
        for b in range(_NBUF):
            g = i * _NBUF + b
            gather(g, b)[1]()
            store(g, b).start()
        for b in range(_NBUF):
            g = i * _NBUF + b
            store(g, b).wait()
            gather(g + _NBUF, b)[0]()
        return carry

    lax.fori_loop(0, _NOUTER - 1, body, 0)

    for b in range(_NBUF):
        g = (_NOUTER - 1) * _NBUF + b
        gather(g, b)[1]()
        store(g, b).start()
    for b in range(_NBUF):
        g = (_NOUTER - 1) * _NBUF + b
        store(g, b).wait()


def kernel(keys, table):
    return _gather_kernel(keys, table)
